# Initial kernel scaffold; baseline (speedup 1.0000x reference)
import jax
import jax.numpy as jnp
from jax.experimental import pallas as pl


def kernel(x, edge_index, W1, a1_src, a1_dst, b1, W2, a2_src, a2_dst, b2):
    raise NotImplementedError("write your pallas kernel here")


# speedup vs baseline: 15.5569x; 15.5569x over previous
import functools

import jax
import jax.numpy as jnp
from jax import lax
from jax.experimental import pallas as pl
from jax.experimental.pallas import tpu as pltpu
from jax.experimental.pallas import tpu_sc as plsc

N = 10000
E = 320000
D = 128
NPAD = 10112
NBLK = NPAD // 128
NTILES = 32
TPT = NPAD
EPAD = NTILES * TPT
CHUNK = 128
NCHUNK = TPT // CHUNK
SUB = 16
ROWS_PER_TILE = NPAD // SUB


def _k1_body(x_ref, w_ref, aa_ref, h_ref, at_ref, m_ref, acc_ref):
    i = pl.program_id(0)
    h = jnp.dot(x_ref[...], w_ref[...], preferred_element_type=jnp.float32)
    row = lax.broadcasted_iota(jnp.int32, (128, 128), 0) + i * 128
    h = jnp.where(row < N, h, 0.0)
    h_ref[...] = h
    at = lax.dot_general(aa_ref[...], h, (((0,), (1,)), ((), ())),
                         preferred_element_type=jnp.float32)
    at_ref[...] = at
    bmax = jnp.max(at[0, :]) + jnp.max(at[1, :])

    @pl.when(i == 0)
    def _():
        acc_ref[0] = 0.0

    acc_ref[0] = jnp.maximum(acc_ref[0], bmax)

    @pl.when(i == NBLK - 1)
    def _():
        m_ref[0, 0] = acc_ref[0]


def _k1(x, w, aa):
    return pl.pallas_call(
        _k1_body,
        grid=(NBLK,),
        in_specs=[
            pl.BlockSpec((128, 128), lambda i: (i, 0)),
            pl.BlockSpec((128, 128), lambda i: (0, 0)),
            pl.BlockSpec((128, 8), lambda i: (0, 0)),
        ],
        out_specs=[
            pl.BlockSpec((128, 128), lambda i: (i, 0)),
            pl.BlockSpec((8, 128), lambda i: (0, i)),
            pl.BlockSpec(memory_space=pltpu.SMEM),
        ],
        out_shape=[
            jax.ShapeDtypeStruct((NPAD, 128), jnp.float32),
            jax.ShapeDtypeStruct((8, NPAD), jnp.float32),
            jax.ShapeDtypeStruct((1, 1), jnp.float32),
        ],
        scratch_shapes=[pltpu.SMEM((1,), jnp.float32)],
    )(x, w, aa)


def _sc_body(src_hbm, dst_hbm, h_hbm, at_hbm, m_hbm,
             num_out, den_out,
             as_v, ad_v, den_v, m_v, src_c, dst_c, w_v, rows, num_sh, sem):
    c = lax.axis_index("c")
    s = lax.axis_index("s")
    wid = c * SUB + s

    pltpu.sync_copy(at_hbm.at[0], as_v)
    pltpu.sync_copy(at_hbm.at[1], ad_v)
    pltpu.sync_copy(m_hbm, m_v)

    zeros16 = jnp.zeros((16,), jnp.float32)

    @pl.loop(0, NPAD // 16)
    def _(j):
        den_v[pl.ds(j * 16, 16)] = zeros16

    @pl.loop(0, CHUNK)
    def _(r):
        for j in range(8):
            rows[r, pl.ds(j * 16, 16)] = zeros16

    base = s * ROWS_PER_TILE
    for part in range(4):
        pltpu.sync_copy(rows, num_sh.at[pl.ds(base + part * 128, 128)])
    pltpu.sync_copy(rows.at[pl.ds(0, ROWS_PER_TILE - 512)],
                    num_sh.at[pl.ds(base + 512, ROWS_PER_TILE - 512)])
    plsc.subcore_barrier()

    mvec = m_v[...]

    @pl.loop(0, NCHUNK)
    def _(k):
        pltpu.sync_copy(src_hbm.at[wid, pl.ds(k * CHUNK, CHUNK)], src_c)
        pltpu.sync_copy(dst_hbm.at[wid, pl.ds(k * CHUNK, CHUNK)], dst_c)
        for j in range(8):
            s16 = src_c[pl.ds(j * 16, 16)]
            d16 = dst_c[pl.ds(j * 16, 16)]
            t = plsc.load_gather(as_v, [s16]) + plsc.load_gather(ad_v, [d16])
            e = jnp.where(t >= 0.0, t, t * jnp.float32(0.2))
            w = jnp.exp(e - mvec)
            w_v[pl.ds(j * 16, 16)] = w
            plsc.addupdate_scatter(den_v, [d16], w)
        pltpu.async_copy(h_hbm.at[src_c], rows, sem).wait()

        @pl.loop(0, CHUNK)
        def _(r):
            wr = plsc.load_gather(w_v, [jnp.full((16,), r, jnp.int32)])
            for j in range(8):
                rows[r, pl.ds(j * 16, 16)] = rows[r, pl.ds(j * 16, 16)] * wr

        pltpu.sync_copy(rows, num_sh.at[dst_c], add=True)

    plsc.subcore_barrier()
    pltpu.sync_copy(den_v, den_out.at[wid])
    for part in range(4):
        pltpu.sync_copy(num_sh.at[pl.ds(base + part * 128, 128)],
                        num_out.at[c, pl.ds(base + part * 128, 128)])
    pltpu.sync_copy(num_sh.at[pl.ds(base + 512, ROWS_PER_TILE - 512)],
                    num_out.at[c, pl.ds(base + 512, ROWS_PER_TILE - 512)])


def _sc_edge(src2, dst2, h, at, mvec):
    mesh = plsc.VectorSubcoreMesh(core_axis_name="c", subcore_axis_name="s")
    fn = pl.kernel(
        _sc_body,
        out_type=[
            jax.ShapeDtypeStruct((2, NPAD, 128), jnp.float32),
            jax.ShapeDtypeStruct((NTILES, NPAD), jnp.float32),
        ],
        mesh=mesh,
        compiler_params=pltpu.CompilerParams(needs_layout_passes=False),
        scratch_types=[
            pltpu.VMEM((NPAD,), jnp.float32),
            pltpu.VMEM((NPAD,), jnp.float32),
            pltpu.VMEM((NPAD,), jnp.float32),
            pltpu.VMEM((16,), jnp.float32),
            pltpu.VMEM((CHUNK,), jnp.int32),
            pltpu.VMEM((CHUNK,), jnp.int32),
            pltpu.VMEM((CHUNK,), jnp.float32),
            pltpu.VMEM((CHUNK, 128), jnp.float32),
            pltpu.VMEM_SHARED((NPAD, 128), jnp.float32),
            pltpu.SemaphoreType.DMA,
        ],
    )
    return fn(src2, dst2, h, at, mvec)


def _k2_body(num_ref, den_ref, b_ref, out_ref):
    ssum = num_ref[0] + num_ref[1]
    d = jnp.sum(den_ref[...], axis=0, keepdims=True)
    dcol = jnp.transpose(jnp.broadcast_to(d, (128, 128)))
    out_ref[...] = ssum / (dcol + 1e-16) + b_ref[...]


def _k2(num, den, b):
    return pl.pallas_call(
        _k2_body,
        grid=(NBLK,),
        in_specs=[
            pl.BlockSpec((2, 128, 128), lambda i: (0, i, 0)),
            pl.BlockSpec((NTILES, 128), lambda i: (0, i)),
            pl.BlockSpec((1, 128), lambda i: (0, 0)),
        ],
        out_specs=pl.BlockSpec((128, 128), lambda i: (i, 0)),
        out_shape=jax.ShapeDtypeStruct((NPAD, 128), jnp.float32),
    )(num, den, b)


def _layer(x, src2, dst2, w, a_src, a_dst, b):
    aa = jnp.zeros((D, 8), jnp.float32)
    aa = aa.at[:, 0].set(a_src).at[:, 1].set(a_dst)
    h, at, m = _k1(x, w, aa)
    mvec = jnp.broadcast_to(m[0, 0], (16,))
    num, den = _sc_edge(src2, dst2, h, at, mvec)
    return _k2(num, den, b.reshape(1, D))


@jax.jit
def kernel(x, edge_index, W1, a1_src, a1_dst, b1, W2, a2_src, a2_dst, b2):
    ei = edge_index.astype(jnp.int32)
    ei = jnp.pad(ei, ((0, 0), (0, EPAD - E)), constant_values=N)
    src2 = ei[0].reshape(NTILES, TPT)
    dst2 = ei[1].reshape(NTILES, TPT)
    h1 = _layer(x, src2, dst2, W1, a1_src, a1_dst, b1)
    h2 = _layer(h1, src2, dst2, W2, a2_src, a2_dst, b2)
    return h2[:N]
